# baseline (speedup 1.0000x reference)
import functools

import jax
import jax.numpy as jnp
from jax import lax
from jax.experimental import pallas as pl
from jax.experimental.pallas import tpu as pltpu
from jax.experimental.pallas import tpu_sc as plsc

N_NODES = 10000
N_EDGES = 320000
D = 128
DH = D // 2

CHUNK = 128
N_CHUNKS = N_EDGES // CHUNK
NC = 2
NS = 16
NCH_BIG = 160
NCH_PIPE15 = 96
NRAG = 4
NBUF = 4
N_PAD = 10112
ROWS_PER_TILE = N_PAD // NS


@functools.partial(
    pl.kernel,
    out_type=jax.ShapeDtypeStruct((NC, N_PAD, DH), jnp.float32),
    mesh=plsc.VectorSubcoreMesh(core_axis_name="c", subcore_axis_name="s"),
    compiler_params=pltpu.CompilerParams(use_tc_tiling_on_sc=False),
    scratch_types=[
        pltpu.VMEM((NCH_BIG * CHUNK,), jnp.int32),
        pltpu.VMEM((NCH_BIG, CHUNK), jnp.int32),
        pltpu.VMEM((CHUNK,), jnp.int32),
        pltpu.VMEM((CHUNK, DH), jnp.float32),
        pltpu.VMEM((CHUNK, DH), jnp.float32),
        pltpu.VMEM((CHUNK, DH), jnp.float32),
        pltpu.VMEM((CHUNK, DH), jnp.float32),
        pltpu.VMEM_SHARED((N_PAD, DH), jnp.float32),
        pltpu.SemaphoreType.DMA,
        pltpu.SemaphoreType.DMA,
        pltpu.SemaphoreType.DMA,
        pltpu.SemaphoreType.DMA,
        pltpu.SemaphoreType.DMA,
    ],
)
def _sc_aggregate(src01_hbm, dst_hbm, dstt_hbm, hh_hbm, zeros_hbm, out_hbm,
                  src_v, dst_v, dstx, rows0, rows1, rows2, rows3, acc_sh,
                  g0, g1, g2, g3, zsem):
    cid = lax.axis_index("c")
    sid = lax.axis_index("s")
    cbase = sid * NCH_BIG

    rows = (rows0, rows1, rows2, rows3)
    gsem = (g0, g1, g2, g3)
    my_slice = acc_sh.at[pl.ds(sid * ROWS_PER_TILE, ROWS_PER_TILE)]

    pltpu.async_copy(zeros_hbm, my_slice, zsem)

    @pl.when(sid < NS - 1)
    def _():
        pltpu.sync_copy(src01_hbm.at[cid, pl.ds(cbase * CHUNK, NCH_BIG * CHUNK)], src_v)
        pltpu.sync_copy(dst_hbm.at[pl.ds(cbase, NCH_BIG)], dst_v)

    @pl.when(sid == NS - 1)
    def _():
        n = NCH_PIPE15 + NRAG
        pltpu.sync_copy(src01_hbm.at[cid, pl.ds(cbase * CHUNK, n * CHUNK)],
                        src_v.at[pl.ds(0, n * CHUNK)])
        pltpu.sync_copy(dst_hbm.at[pl.ds(cbase, NCH_PIPE15)],
                        dst_v.at[pl.ds(0, NCH_PIPE15)])

    nq = jnp.where(sid == NS - 1, NCH_PIPE15 // NBUF, NCH_BIG // NBUF)
    nch = jnp.where(sid == NS - 1, NCH_PIPE15, NCH_BIG)

    def fire_gather(c, b):
        pltpu.async_copy(
            hh_hbm.at[src_v.at[pl.ds(c * CHUNK, CHUNK)]], rows[b], gsem[b])

    def wait_gather(b):
        pltpu.make_async_copy(
            hh_hbm.at[src_v.at[pl.ds(0, CHUNK)]], rows[b], gsem[b]).wait()

    for b in range(NBUF - 1):
        fire_gather(b, b)

    pltpu.make_async_copy(zeros_hbm, my_slice, zsem).wait()
    plsc.subcore_barrier()

    def body(j, carry):
        for p in range(NBUF):
            c = NBUF * j + p
            pn = (p + NBUF - 1) % NBUF

            @pl.when(c + NBUF - 1 < nch)
            def _():
                fire_gather(c + NBUF - 1, pn)

            wait_gather(p)
            pltpu.sync_copy(rows[p], acc_sh.at[dst_v.at[c]], add=True)
        return carry

    lax.fori_loop(0, nq, body, 0)

    @pl.when(sid == NS - 1)
    def _():
        def rag(t, carry):
            pltpu.sync_copy(dstt_hbm.at[pl.ds(t * CHUNK, CHUNK)], dstx)
            pltpu.async_copy(
                hh_hbm.at[src_v.at[pl.ds((NCH_PIPE15 + t) * CHUNK, CHUNK)]],
                rows0, g0).wait()
            pltpu.sync_copy(rows0, acc_sh.at[dstx], add=True)
            return carry

        lax.fori_loop(0, NRAG, rag, 0)

    plsc.subcore_barrier()

    pltpu.sync_copy(
        my_slice,
        out_hbm.at[cid, pl.ds(sid * ROWS_PER_TILE, ROWS_PER_TILE)],
    )


BLK = 2000


def _mlp_body(scale_ref, h_ref, p_ref, w1_ref, b1_ref, w2_ref, b2_ref, o_ref):
    scale = scale_ref[0]
    agg = jnp.concatenate([p_ref[0], p_ref[1]], axis=1)
    z = h_ref[...] * scale + agg
    z = jnp.dot(z, w1_ref[...], preferred_element_type=jnp.float32) + b1_ref[...]
    z = jnp.maximum(z, 0.0)
    z = jnp.dot(z, w2_ref[...], preferred_element_type=jnp.float32) + b2_ref[...]
    o_ref[...] = jnp.maximum(z, 0.0)


_tc_mlp = pl.pallas_call(
    _mlp_body,
    grid=(N_NODES // BLK,),
    in_specs=[
        pl.BlockSpec(memory_space=pltpu.SMEM),
        pl.BlockSpec((BLK, D), lambda i: (i, 0)),
        pl.BlockSpec((NC, BLK, DH), lambda i: (0, i, 0)),
        pl.BlockSpec((D, D), lambda i: (0, 0)),
        pl.BlockSpec((1, D), lambda i: (0, 0)),
        pl.BlockSpec((D, D), lambda i: (0, 0)),
        pl.BlockSpec((1, D), lambda i: (0, 0)),
    ],
    out_specs=pl.BlockSpec((BLK, D), lambda i: (i, 0)),
    out_shape=jax.ShapeDtypeStruct((N_NODES, D), jnp.float32),
)


def _gin_layer(h, src01, dst2d, dstt, zeros, eps, W1, b1, W2, b2):
    hh = jnp.concatenate([h[:, :DH], h[:, DH:]], axis=0)
    parts = _sc_aggregate(src01, dst2d, dstt, hh, zeros)
    scale = (1.0 + eps).reshape((1,)).astype(jnp.float32)
    return _tc_mlp(scale, h, parts, W1, b1.reshape(1, D), W2, b2.reshape(1, D))


def kernel(x, edge_index, eps0, W1_0, b1_0, W2_0, b2_0, eps1, W1_1, b1_1, W2_1, b2_1):
    src = edge_index[0]
    dst1 = edge_index[1]
    src01 = jnp.stack([src, src + N_NODES])
    dst2d = dst1.reshape(N_CHUNKS, CHUNK)
    dstt = dst1[(N_CHUNKS - NRAG) * CHUNK:]
    zeros = jnp.zeros((ROWS_PER_TILE, DH), jnp.float32)
    h = _gin_layer(x, src01, dst2d, dstt, zeros, eps0, W1_0, b1_0, W2_0, b2_0)
    h = _gin_layer(h, src01, dst2d, dstt, zeros, eps1, W1_1, b1_1, W2_1, b2_1)
    return h


# speedup vs baseline: 1.1894x; 1.1894x over previous
import functools

import jax
import jax.numpy as jnp
from jax import lax
from jax.experimental import pallas as pl
from jax.experimental.pallas import tpu as pltpu
from jax.experimental.pallas import tpu_sc as plsc

N_NODES = 10000
N_EDGES = 320000
D = 128

CHUNK = 128
NC = 2
NS = 16
NW = NC * NS
EPW = N_EDGES // NW
NCH = EPW // CHUNK
TAIL = EPW - NCH * CHUNK
N_PAD = 10112
ROWS_PER_TILE = N_PAD // NS
NZ = 5


@functools.partial(
    pl.kernel,
    out_type=jax.ShapeDtypeStruct((NC, N_PAD, D), jnp.float32),
    mesh=plsc.VectorSubcoreMesh(core_axis_name="c", subcore_axis_name="s"),
    scratch_types=[
        pltpu.VMEM((EPW,), jnp.int32),
        pltpu.VMEM((CHUNK,), jnp.int32),
        pltpu.VMEM((CHUNK,), jnp.int32),
        pltpu.VMEM((TAIL,), jnp.int32),
        pltpu.VMEM((CHUNK, D), jnp.float32),
        pltpu.VMEM((CHUNK, D), jnp.float32),
        pltpu.VMEM_SHARED((N_PAD, D), jnp.float32),
        pltpu.SemaphoreType.DMA,
        pltpu.SemaphoreType.DMA,
        pltpu.SemaphoreType.DMA,
        pltpu.SemaphoreType.DMA,
        pltpu.SemaphoreType.DMA,
        pltpu.SemaphoreType.DMA,
    ],
)
def _sc_aggregate(src_hbm, dst_hbm, h_hbm, out_hbm,
                  src_v, dst0, dst1, dstt, rows0, rows1, acc_sh,
                  gsem0, gsem1, dsem0, dsem1, zsem, tsem):
    cid = lax.axis_index("c")
    sid = lax.axis_index("s")
    wid = sid * NC + cid
    ebase = wid * EPW

    zvec = jnp.zeros((16,), jnp.float32)

    def zfill(r, carry):
        for q in range(8):
            rows1[r, pl.ds(q * 16, 16)] = zvec
        return carry

    lax.fori_loop(0, CHUNK, zfill, 0)

    for k in range(NZ - 1):
        pltpu.async_copy(
            rows1, acc_sh.at[pl.ds(sid * ROWS_PER_TILE + k * CHUNK, CHUNK)], zsem)
    last = ROWS_PER_TILE - (NZ - 1) * CHUNK
    pltpu.async_copy(
        rows1.at[pl.ds(0, last)],
        acc_sh.at[pl.ds(sid * ROWS_PER_TILE + (NZ - 1) * CHUNK, last)], zsem)

    pltpu.sync_copy(src_hbm.at[pl.ds(ebase, EPW)], src_v)

    dsts = (dst0, dst1)
    dsems = (dsem0, dsem1)
    rows = (rows0, rows1)
    gsems = (gsem0, gsem1)

    def fire_dst(c, b):
        pltpu.async_copy(dst_hbm.at[pl.ds(ebase + c * CHUNK, CHUNK)], dsts[b], dsems[b])

    def fire_gather(c, b):
        pltpu.async_copy(h_hbm.at[src_v.at[pl.ds(c * CHUNK, CHUNK)]], rows[b], gsems[b])

    def wait_dst(c, b):
        pltpu.make_async_copy(
            dst_hbm.at[pl.ds(ebase + c * CHUNK, CHUNK)], dsts[b], dsems[b]).wait()

    def wait_gather(c, b):
        pltpu.make_async_copy(
            h_hbm.at[src_v.at[pl.ds(c * CHUNK, CHUNK)]], rows[b], gsems[b]).wait()

    fire_dst(0, 0)
    fire_dst(1, 1)
    fire_gather(0, 0)
    pltpu.async_copy(dst_hbm.at[pl.ds(ebase + NCH * CHUNK, TAIL)], dstt, tsem)

    for k in range(NZ - 1):
        pltpu.make_async_copy(
            rows1, acc_sh.at[pl.ds(sid * ROWS_PER_TILE, CHUNK)], zsem).wait()
    pltpu.make_async_copy(
        rows1.at[pl.ds(0, last)],
        acc_sh.at[pl.ds(sid * ROWS_PER_TILE, last)], zsem).wait()
    plsc.subcore_barrier()

    def body(j, carry):
        c0 = 2 * j
        c1 = 2 * j + 1
        fire_gather(c1, 1)
        wait_gather(c0, 0)
        wait_dst(c0, 0)
        pltpu.sync_copy(rows0, acc_sh.at[dst0], add=True)

        @pl.when(c1 + 1 < NCH)
        def _():
            fire_dst(c0 + 2, 0)
            fire_gather(c1 + 1, 0)

        wait_gather(c1, 1)
        wait_dst(c1, 1)
        pltpu.sync_copy(rows1, acc_sh.at[dst1], add=True)

        @pl.when(c1 + 2 < NCH)
        def _():
            fire_dst(c1 + 2, 1)

        return carry

    lax.fori_loop(0, NCH // 2, body, 0)

    pltpu.make_async_copy(
        dst_hbm.at[pl.ds(ebase + NCH * CHUNK, TAIL)], dstt, tsem).wait()
    pltpu.async_copy(
        h_hbm.at[src_v.at[pl.ds(NCH * CHUNK, TAIL)]], rows0.at[pl.ds(0, TAIL)], gsem0
    ).wait()
    pltpu.sync_copy(rows0.at[pl.ds(0, TAIL)], acc_sh.at[dstt], add=True)

    plsc.subcore_barrier()

    pltpu.sync_copy(
        acc_sh.at[pl.ds(sid * ROWS_PER_TILE, ROWS_PER_TILE)],
        out_hbm.at[cid, pl.ds(sid * ROWS_PER_TILE, ROWS_PER_TILE)],
    )


BLK = 2000


def _mlp_body(scale_ref, h_ref, p_ref, w1_ref, b1_ref, w2_ref, b2_ref, o_ref):
    scale = scale_ref[0]
    z = h_ref[...] * scale + p_ref[0] + p_ref[1]
    z = jnp.dot(z, w1_ref[...], preferred_element_type=jnp.float32) + b1_ref[...]
    z = jnp.maximum(z, 0.0)
    z = jnp.dot(z, w2_ref[...], preferred_element_type=jnp.float32) + b2_ref[...]
    o_ref[...] = jnp.maximum(z, 0.0)


_tc_mlp = pl.pallas_call(
    _mlp_body,
    grid=(N_NODES // BLK,),
    in_specs=[
        pl.BlockSpec(memory_space=pltpu.SMEM),
        pl.BlockSpec((BLK, D), lambda i: (i, 0)),
        pl.BlockSpec((NC, BLK, D), lambda i: (0, i, 0)),
        pl.BlockSpec((D, D), lambda i: (0, 0)),
        pl.BlockSpec((1, D), lambda i: (0, 0)),
        pl.BlockSpec((D, D), lambda i: (0, 0)),
        pl.BlockSpec((1, D), lambda i: (0, 0)),
    ],
    out_specs=pl.BlockSpec((BLK, D), lambda i: (i, 0)),
    out_shape=jax.ShapeDtypeStruct((N_NODES, D), jnp.float32),
)


def _gin_layer(h, src, dst, eps, W1, b1, W2, b2):
    parts = _sc_aggregate(src, dst, h)
    scale = (1.0 + eps).reshape((1,)).astype(jnp.float32)
    return _tc_mlp(scale, h, parts, W1, b1.reshape(1, D), W2, b2.reshape(1, D))


def kernel(x, edge_index, eps0, W1_0, b1_0, W2_0, b2_0, eps1, W1_1, b1_1, W2_1, b2_1):
    src = edge_index[0]
    dst = edge_index[1]
    h = _gin_layer(x, src, dst, eps0, W1_0, b1_0, W2_0, b2_0)
    h = _gin_layer(h, src, dst, eps1, W1_1, b1_1, W2_1, b2_1)
    return h
